# baseline (speedup 1.0000x reference)
import functools
import jax
import jax.numpy as jnp
import numpy as np
from jax import lax
from jax.experimental import pallas as pl
from jax.experimental.pallas import tpu as pltpu
from jax.experimental.pallas import tpu_sc as plsc

_B, _S, _V = 64, 8, 100000
_NC, _NS, _L = 2, 16, 16
_NW = _NC * _NS
_RPW = _B // _NW
_C = 10000
_NCH = _V // _C
_NK = _RPW * _NCH
_U = 5
_NIT = _C // _L // _U

_G = np.asarray(jax.random.gumbel(jax.random.key(42), (_B, _V), jnp.float32))

_mesh = plsc.VectorSubcoreMesh(core_axis_name="c", subcore_axis_name="s")


def _shuffle(x, idx):
    return lax.gather(
        x, idx[:, None],
        lax.GatherDimensionNumbers(offset_dims=(), collapsed_slice_dims=(0,),
                                   start_index_map=(0,)),
        slice_sizes=(1,), mode=lax.GatherScatterMode.PROMISE_IN_BOUNDS)


def _merge(v1, i1, v2, i2):
    take = (v2 > v1) | ((v2 == v1) & (i2 < i1))
    return jnp.where(take, v2, v1), jnp.where(take, i2, i1)


def _lane_argmax(v, i, lanes):
    for off in (8, 4, 2, 1):
        perm = lanes ^ off
        v, i = _merge(v, i, _shuffle(v, perm), _shuffle(i, perm))
    return i


@functools.partial(
    pl.kernel, mesh=_mesh,
    out_type=[
        jax.ShapeDtypeStruct((_B * _V,), jnp.float32),
        jax.ShapeDtypeStruct((_NW * _L,), jnp.int32),
    ],
    scratch_types=[
        pltpu.VMEM((_C,), jnp.float32),
        pltpu.VMEM((_C,), jnp.float32),
        pltpu.VMEM((_C,), jnp.float32),
        pltpu.VMEM((_C,), jnp.float32),
        pltpu.VMEM((_C,), jnp.float32),
        pltpu.VMEM((_C,), jnp.float32),
        pltpu.VMEM((_C,), jnp.float32),
        pltpu.VMEM((_C,), jnp.float32),
        pltpu.VMEM((_L,), jnp.int32),
        pltpu.SemaphoreType.DMA,
        pltpu.SemaphoreType.DMA,
        pltpu.SemaphoreType.DMA,
        pltpu.SemaphoreType.DMA,
    ],
)
def _sc_kern(logits1d, g_hbm, mask_hbm, masked_hbm, ids_hbm,
             l0, l1, g0, g1, m0, m1, o0, o1, ibuf,
             sin0, sin1, sout0, sout1):
    cid = lax.axis_index("c")
    sid = lax.axis_index("s")
    wid = sid * _NC + cid
    lanes = lax.iota(jnp.int32, _L)

    def in_descs(k, lb, gb, mb, sem):
        row = wid * _RPW + k // _NCH
        off = (k % _NCH) * _C
        return (
            pltpu.make_async_copy(
                logits1d.at[pl.ds((row * _S + _S - 1) * _V + off, _C)],
                lb, sem),
            pltpu.make_async_copy(g_hbm.at[pl.ds(row * _V + off, _C)],
                                  gb, sem),
            pltpu.make_async_copy(mask_hbm.at[pl.ds(off, _C)], mb, sem),
        )

    def out_desc(k, ob, sem):
        row = wid * _RPW + k // _NCH
        off = (k % _NCH) * _C
        return pltpu.make_async_copy(
            ob, masked_hbm.at[pl.ds(row * _V + off, _C)], sem)

    def in_start(k, lb, gb, mb, sem):
        for d in in_descs(k, lb, gb, mb, sem):
            d.start()

    def in_wait(k, lb, gb, mb, sem):
        for d in in_descs(k, lb, gb, mb, sem):
            d.wait()

    def compute(k, lb, gb, mb, ob, bv0, bi0, bv1, bi1):
        row_sel = k // _NCH
        colbase = (k % _NCH) * _C

        ninf = jnp.full((_L,), -jnp.inf, jnp.float32)
        zero = jnp.zeros((_L,), jnp.int32)
        acc = [(ninf, zero)] * _U

        def vbody(i, c):
            accs = list(zip(c[0::2], c[1::2]))
            out = []
            base = i * (_U * _L)
            for u in range(_U):
                sl = pl.ds(base + u * _L, _L)
                v = lb[sl] + mb[sl]
                ob[sl] = v
                t = v + gb[sl]
                idxv = lanes + (colbase + base + u * _L)
                bv, bi = accs[u]
                upd = t > bv
                out.append(jnp.where(upd, t, bv))
                out.append(jnp.where(upd, idxv, bi))
            return tuple(out)

        flat = tuple(x for p in acc for x in p)
        flat = lax.fori_loop(0, _NIT, vbody, flat)
        mv, mi = flat[0], flat[1]
        for u in range(1, _U):
            mv, mi = _merge(mv, mi, flat[2 * u], flat[2 * u + 1])

        is_r0 = row_sel == 0
        nbv0, nbi0 = _merge(bv0, bi0, jnp.where(is_r0, mv, -jnp.inf),
                            jnp.where(is_r0, mi, 0))
        nbv1, nbi1 = _merge(bv1, bi1, jnp.where(is_r0, -jnp.inf, mv),
                            jnp.where(is_r0, 0, mi))
        return nbv0, nbi0, nbv1, nbi1

    in_start(0, l0, g0, m0, sin0)

    ninf = jnp.full((_L,), -jnp.inf, jnp.float32)
    zero = jnp.zeros((_L,), jnp.int32)

    def obody(o, carry):
        bv0, bi0, bv1, bi1 = carry
        k0 = 2 * o
        k1 = k0 + 1

        in_start(k1, l1, g1, m1, sin1)
        in_wait(k0, l0, g0, m0, sin0)

        @pl.when(o > 0)
        def _drain0():
            out_desc(k0, o0, sout0).wait()

        bv0, bi0, bv1, bi1 = compute(k0, l0, g0, m0, o0, bv0, bi0, bv1, bi1)
        out_desc(k0, o0, sout0).start()

        @pl.when(k0 + 2 < _NK)
        def _next0():
            in_start(k0 + 2, l0, g0, m0, sin0)

        in_wait(k1, l1, g1, m1, sin1)

        @pl.when(o > 0)
        def _drain1():
            out_desc(k1, o1, sout1).wait()

        bv0, bi0, bv1, bi1 = compute(k1, l1, g1, m1, o1, bv0, bi0, bv1, bi1)
        out_desc(k1, o1, sout1).start()
        return bv0, bi0, bv1, bi1

    bv0, bi0, bv1, bi1 = lax.fori_loop(
        0, _NK // 2, obody, (ninf, zero, ninf, zero))

    out_desc(_NK - 2, o0, sout0).wait()
    out_desc(_NK - 1, o1, sout1).wait()

    rid0 = _lane_argmax(bv0, bi0, lanes)
    rid1 = _lane_argmax(bv1, bi1, lanes)
    ibuf[...] = jnp.where(lanes == 0, rid0,
                          jnp.where(lanes == 1, rid1, 0))
    pltpu.sync_copy(ibuf, ids_hbm.at[pl.ds(wid * _L, _L)])


def kernel(predicted_logits, prediction_mask):
    logits1d = predicted_logits.reshape(_B * _S * _V)
    masked, ids_pad = _sc_kern(logits1d, jnp.asarray(_G).reshape(_B * _V),
                               prediction_mask)
    ids = ids_pad.reshape(_NW, _L)[:, :_RPW].reshape(_B)
    return ids, masked.reshape(_B, _V)


# speedup vs baseline: 10.6435x; 10.6435x over previous
import jax
import jax.numpy as jnp
import numpy as np
from jax.experimental import pallas as pl
from jax.experimental.pallas import tpu as pltpu

_B, _S, _V = 64, 8, 100000
_VT = 16384
_NV = (_V + _VT - 1) // _VT
_TAIL = _V - (_NV - 1) * _VT

_G = np.asarray(jax.random.gumbel(jax.random.key(42), (_B, _V), jnp.float32))


def _body(logits_hbm, tail_hbm, mask_ref, g_ref, masked_ref, ids_ref,
          lbuf, sem, best_val, best_idx):
    j = pl.program_id(0)

    def start_main(k):
        pltpu.make_async_copy(
            logits_hbm.at[:, _S - 1, pl.ds(k * _VT, _VT)],
            lbuf.at[jax.lax.rem(k, 2)], sem.at[jax.lax.rem(k, 2)]).start()

    @pl.when(j == 0)
    def _prime():
        start_main(0)

    @pl.when(j + 1 < _NV - 1)
    def _next_main():
        start_main(j + 1)

    @pl.when(j + 1 == _NV - 1)
    def _next_tail():
        slot = jax.lax.rem(_NV - 1, 2)
        pltpu.make_async_copy(tail_hbm, lbuf.at[slot], sem.at[slot]).start()

    slot = jax.lax.rem(j, 2)
    pltpu.make_async_copy(
        logits_hbm.at[:, _S - 1, pl.ds(0, _VT)],
        lbuf.at[slot], sem.at[slot]).wait()

    vals = lbuf[slot] + mask_ref[0, :][None, :]
    masked_ref[...] = vals
    tot = vals + g_ref[...]
    col = jax.lax.broadcasted_iota(jnp.int32, (_B, _VT), 1) + j * _VT
    tot = jnp.where(col < _V, tot, -jnp.inf)
    bmax = jnp.max(tot, axis=1)[:, None]
    bidx = jnp.argmax(tot, axis=1)[:, None] + j * _VT

    @pl.when(j == 0)
    def _init():
        best_val[...] = bmax
        best_idx[...] = bidx

    @pl.when(j > 0)
    def _acc():
        upd = bmax > best_val[...]
        best_val[...] = jnp.where(upd, bmax, best_val[...])
        best_idx[...] = jnp.where(upd, bidx, best_idx[...])

    @pl.when(j == _NV - 1)
    def _emit():
        ids_ref[...] = best_idx[...]


def kernel(predicted_logits, prediction_mask):
    mask2d = prediction_mask.reshape(1, _V)
    tail = jnp.pad(predicted_logits[:, -1, (_NV - 1) * _VT:],
                   ((0, 0), (0, _VT - _TAIL)))
    masked, ids = pl.pallas_call(
        _body,
        grid=(_NV,),
        in_specs=[
            pl.BlockSpec(memory_space=pltpu.MemorySpace.HBM),
            pl.BlockSpec(memory_space=pltpu.MemorySpace.HBM),
            pl.BlockSpec((1, _VT), lambda j: (0, j)),
            pl.BlockSpec((_B, _VT), lambda j: (0, j)),
        ],
        out_specs=[
            pl.BlockSpec((_B, _VT), lambda j: (0, j)),
            pl.BlockSpec((_B, 1), lambda j: (0, 0)),
        ],
        out_shape=[
            jax.ShapeDtypeStruct((_B, _V), jnp.float32),
            jax.ShapeDtypeStruct((_B, 1), jnp.int32),
        ],
        scratch_shapes=[
            pltpu.VMEM((2, _B, _VT), jnp.float32),
            pltpu.SemaphoreType.DMA((2,)),
            pltpu.VMEM((_B, 1), jnp.float32),
            pltpu.VMEM((_B, 1), jnp.int32),
        ],
    )(predicted_logits, tail, mask2d, jnp.asarray(_G))
    return ids[:, 0], masked


# speedup vs baseline: 11.0458x; 1.0378x over previous
import jax
import jax.numpy as jnp
import numpy as np
from jax.experimental import pallas as pl
from jax.experimental.pallas import tpu as pltpu

_B, _S, _V = 64, 8, 100000
_VT = 8192
_NV = (_V + _VT - 1) // _VT
_TAIL = _V - (_NV - 1) * _VT

_G = np.asarray(jax.random.gumbel(jax.random.key(42), (_B, _V), jnp.float32))


def _body(logits_hbm, tail_hbm, mask_ref, g_ref, masked_ref, ids_ref,
          lbuf, sem, best_val, best_idx):
    j = pl.program_id(0)

    def start_main(k):
        pltpu.make_async_copy(
            logits_hbm.at[:, _S - 1, pl.ds(k * _VT, _VT)],
            lbuf.at[jax.lax.rem(k, 2)], sem.at[jax.lax.rem(k, 2)]).start()

    @pl.when(j == 0)
    def _prime():
        start_main(0)

    @pl.when(j + 1 < _NV - 1)
    def _next_main():
        start_main(j + 1)

    @pl.when(j + 1 == _NV - 1)
    def _next_tail():
        slot = jax.lax.rem(_NV - 1, 2)
        pltpu.make_async_copy(tail_hbm, lbuf.at[slot], sem.at[slot]).start()

    slot = jax.lax.rem(j, 2)
    pltpu.make_async_copy(
        logits_hbm.at[:, _S - 1, pl.ds(0, _VT)],
        lbuf.at[slot], sem.at[slot]).wait()

    vals = lbuf[slot] + mask_ref[0, :][None, :]
    masked_ref[...] = vals
    tot = vals + g_ref[...]
    col = jax.lax.broadcasted_iota(jnp.int32, (_B, _VT), 1) + j * _VT
    tot = jnp.where(col < _V, tot, -jnp.inf)
    bmax = jnp.max(tot, axis=1)[:, None]
    bidx = jnp.argmax(tot, axis=1)[:, None] + j * _VT

    @pl.when(j == 0)
    def _init():
        best_val[...] = bmax
        best_idx[...] = bidx

    @pl.when(j > 0)
    def _acc():
        upd = bmax > best_val[...]
        best_val[...] = jnp.where(upd, bmax, best_val[...])
        best_idx[...] = jnp.where(upd, bidx, best_idx[...])

    @pl.when(j == _NV - 1)
    def _emit():
        ids_ref[...] = best_idx[...]


def kernel(predicted_logits, prediction_mask):
    mask2d = prediction_mask.reshape(1, _V)
    tail = jnp.pad(predicted_logits[:, -1, (_NV - 1) * _VT:],
                   ((0, 0), (0, _VT - _TAIL)))
    masked, ids = pl.pallas_call(
        _body,
        grid=(_NV,),
        in_specs=[
            pl.BlockSpec(memory_space=pltpu.MemorySpace.HBM),
            pl.BlockSpec(memory_space=pltpu.MemorySpace.HBM),
            pl.BlockSpec((1, _VT), lambda j: (0, j)),
            pl.BlockSpec((_B, _VT), lambda j: (0, j)),
        ],
        out_specs=[
            pl.BlockSpec((_B, _VT), lambda j: (0, j)),
            pl.BlockSpec((_B, 1), lambda j: (0, 0)),
        ],
        out_shape=[
            jax.ShapeDtypeStruct((_B, _V), jnp.float32),
            jax.ShapeDtypeStruct((_B, 1), jnp.int32),
        ],
        scratch_shapes=[
            pltpu.VMEM((2, _B, _VT), jnp.float32),
            pltpu.SemaphoreType.DMA((2,)),
            pltpu.VMEM((_B, 1), jnp.float32),
            pltpu.VMEM((_B, 1), jnp.int32),
        ],
    )(predicted_logits, tail, mask2d, jnp.asarray(_G))
    return ids[:, 0], masked


# speedup vs baseline: 12.7140x; 1.1510x over previous
import jax
import jax.numpy as jnp
import numpy as np
from jax.experimental import pallas as pl
from jax.experimental.pallas import tpu as pltpu

_B, _S, _V = 64, 8, 100000
_VT = 8192
_NV = (_V + _VT - 1) // _VT
_TAIL = _V - (_NV - 1) * _VT
_NSLOT = 4

_G = np.asarray(jax.random.gumbel(jax.random.key(42), (_B, _V), jnp.float32))


def _body(logits_hbm, tail_hbm, mask_ref, g_ref, masked_ref, ids_ref,
          lbuf, sem, best_val, best_idx):
    j = pl.program_id(0)

    def start(k):
        slot = jax.lax.rem(k, _NSLOT)

        @pl.when(k < _NV - 1)
        def _main():
            pltpu.make_async_copy(
                logits_hbm.at[:, _S - 1, pl.ds(k * _VT, _VT)],
                lbuf.at[slot], sem.at[slot]).start()

        @pl.when(k == _NV - 1)
        def _tail():
            pltpu.make_async_copy(tail_hbm, lbuf.at[slot],
                                  sem.at[slot]).start()

    @pl.when(j == 0)
    def _prime():
        for k in range(_NSLOT - 1):
            start(k)

    @pl.when(j + _NSLOT - 1 < _NV)
    def _ahead():
        start(j + _NSLOT - 1)

    slot = jax.lax.rem(j, _NSLOT)
    pltpu.make_async_copy(
        logits_hbm.at[:, _S - 1, pl.ds(0, _VT)],
        lbuf.at[slot], sem.at[slot]).wait()

    vals = lbuf[slot] + mask_ref[...][None, :]
    masked_ref[...] = vals
    tot = vals + g_ref[...]
    col = jax.lax.broadcasted_iota(jnp.int32, (_B, _VT), 1) + j * _VT
    tot = jnp.where(col < _V, tot, -jnp.inf)
    bmax = jnp.max(tot, axis=1)[:, None]
    bidx = jnp.argmax(tot, axis=1)[:, None] + j * _VT

    @pl.when(j == 0)
    def _init():
        best_val[...] = bmax
        best_idx[...] = bidx

    @pl.when(j > 0)
    def _acc():
        upd = bmax > best_val[...]
        best_val[...] = jnp.where(upd, bmax, best_val[...])
        best_idx[...] = jnp.where(upd, bidx, best_idx[...])

    @pl.when(j == _NV - 1)
    def _emit():
        ids_ref[...] = best_idx[...]


def kernel(predicted_logits, prediction_mask):
    tail = jnp.pad(predicted_logits[:, -1, (_NV - 1) * _VT:],
                   ((0, 0), (0, _VT - _TAIL)))
    masked, ids = pl.pallas_call(
        _body,
        grid=(_NV,),
        in_specs=[
            pl.BlockSpec(memory_space=pltpu.MemorySpace.HBM),
            pl.BlockSpec(memory_space=pltpu.MemorySpace.HBM),
            pl.BlockSpec((_VT,), lambda j: (j,)),
            pl.BlockSpec((_B, _VT), lambda j: (0, j)),
        ],
        out_specs=[
            pl.BlockSpec((_B, _VT), lambda j: (0, j)),
            pl.BlockSpec((_B, 1), lambda j: (0, 0)),
        ],
        out_shape=[
            jax.ShapeDtypeStruct((_B, _V), jnp.float32),
            jax.ShapeDtypeStruct((_B, 1), jnp.int32),
        ],
        scratch_shapes=[
            pltpu.VMEM((_NSLOT, _B, _VT), jnp.float32),
            pltpu.SemaphoreType.DMA((_NSLOT,)),
            pltpu.VMEM((_B, 1), jnp.float32),
            pltpu.VMEM((_B, 1), jnp.int32),
        ],
    )(predicted_logits, tail, prediction_mask, jnp.asarray(_G))
    return ids[:, 0], masked
